# Initial kernel scaffold; baseline (speedup 1.0000x reference)
import jax
import jax.numpy as jnp
from jax.experimental import pallas as pl


def kernel(xs, eis, W1, b1, Wc1, bc1, Wc2, bc2, W_ih, W_hh, b_ih, b_hh, Wl, bl):
    raise NotImplementedError("write your pallas kernel here")


# speedup vs baseline: 26.4250x; 26.4250x over previous
import functools

import jax
import jax.numpy as jnp
from jax import lax
from jax.experimental import pallas as pl
from jax.experimental.pallas import tpu as pltpu
from jax.experimental.pallas import tpu_sc as plsc

N = 10000
T = 12
E = 320000
X_DIM = 128
H = 32
Z = 16

NC = 2
NS = 16
TPC = T // NC
EPT = E // NS
C = 80
NCH = EPT // C
RPT = N // NS
DW = 16

_sc_mesh = plsc.VectorSubcoreMesh(core_axis_name="c", subcore_axis_name="s")
_sc_params = pltpu.CompilerParams(use_tc_tiling_on_sc=False)


@functools.partial(
    pl.kernel,
    out_type=jax.ShapeDtypeStruct((T, N, DW), jnp.float32),
    mesh=_sc_mesh,
    compiler_params=_sc_params,
    scratch_types=[
        pltpu.VMEM((NCH, C), jnp.int32),
        pltpu.VMEM((RPT, DW), jnp.float32),
        pltpu.VMEM_SHARED((N, DW), jnp.float32),
    ],
)
def _sc_deg(dst_hbm, ones_hbm, out_hbm, dst_v, ones_v, acc):
    c = lax.axis_index("c")
    s = lax.axis_index("s")
    rows = pl.ds(s * RPT, RPT)
    pltpu.sync_copy(ones_hbm, ones_v)

    def t_body(k, carry):
        t = c * TPC + k
        pltpu.sync_copy(dst_hbm.at[t, s], dst_v)
        pltpu.sync_copy(ones_v, acc.at[rows])
        plsc.subcore_barrier()

        def ch_body(j, carry2):
            pltpu.sync_copy(ones_v.at[pl.ds(0, C)], acc.at[dst_v.at[j]],
                            add=True)
            return carry2

        lax.fori_loop(0, NCH, ch_body, 0)
        plsc.subcore_barrier()
        pltpu.sync_copy(acc.at[rows], out_hbm.at[t, rows])
        return carry

    lax.fori_loop(0, TPC, t_body, 0)


@functools.partial(
    pl.kernel,
    out_type=jax.ShapeDtypeStruct((T, N, H), jnp.float32),
    mesh=_sc_mesh,
    compiler_params=_sc_params,
    scratch_types=[
        pltpu.VMEM((NCH, C), jnp.int32),
        pltpu.VMEM((NCH, C), jnp.int32),
        pltpu.VMEM((C, H), jnp.float32),
        pltpu.VMEM_SHARED((N, H), jnp.float32),
        pltpu.SemaphoreType.DMA,
    ],
)
def _sc_agg(src_hbm, dst_hbm, g_hbm, out_hbm, src_v, dst_v, rows_v, acc, sem):
    c = lax.axis_index("c")
    s = lax.axis_index("s")
    rows = pl.ds(s * RPT, RPT)

    def t_body(k, carry):
        t = c * TPC + k
        pltpu.sync_copy(src_hbm.at[t, s], src_v)
        pltpu.sync_copy(dst_hbm.at[t, s], dst_v)
        g_t = g_hbm.at[t]
        pltpu.sync_copy(g_t.at[rows], acc.at[rows])
        plsc.subcore_barrier()

        def ch_body(j, carry2):
            pltpu.async_copy(g_t.at[src_v.at[j]], rows_v, sem).wait()
            pltpu.sync_copy(rows_v, acc.at[dst_v.at[j]], add=True)
            return carry2

        lax.fori_loop(0, NCH, ch_body, 0)
        plsc.subcore_barrier()
        pltpu.sync_copy(acc.at[rows], out_hbm.at[t, rows])
        return carry

    lax.fori_loop(0, TPC, t_body, 0)


def _k_h_body(xs_ref, w_ref, b_ref, o_ref):
    o_ref[...] = jnp.maximum(
        jnp.dot(xs_ref[...], w_ref[...], preferred_element_type=jnp.float32)
        + b_ref[...], 0.0)


def _k_h(xs, W1, b1r):
    return pl.pallas_call(
        _k_h_body,
        out_shape=jax.ShapeDtypeStruct((N, H), jnp.float32),
    )(xs, W1, b1r)


_BN = 1000


def _k_g1_body(deg_ref, h_ref, dinv_ref, g1_ref):
    d = deg_ref[0]
    d32 = jnp.concatenate([d, d], axis=-1)
    dinv = lax.rsqrt(d32)
    dinv_ref[0] = dinv
    g1_ref[0] = h_ref[...] * dinv


def _k_g1(deg16, h):
    return pl.pallas_call(
        _k_g1_body,
        grid=(T, N // _BN),
        in_specs=[
            pl.BlockSpec((1, _BN, DW), lambda t, nb: (t, nb, 0)),
            pl.BlockSpec((_BN, H), lambda t, nb: (nb, 0)),
        ],
        out_specs=[
            pl.BlockSpec((1, _BN, H), lambda t, nb: (t, nb, 0)),
            pl.BlockSpec((1, _BN, H), lambda t, nb: (t, nb, 0)),
        ],
        out_shape=[
            jax.ShapeDtypeStruct((T, N, H), jnp.float32),
            jax.ShapeDtypeStruct((T, N, H), jnp.float32),
        ],
    )(deg16, h)


_BM = 8000


def _k_mid_body(a_ref, dinv_ref, w_ref, b_ref, o_ref):
    dv = dinv_ref[...]
    y = jnp.dot(a_ref[...], w_ref[...], preferred_element_type=jnp.float32)
    o_ref[...] = jnp.maximum(dv * y + b_ref[...], 0.0) * dv


def _k_mid(A1f, dinvf, Wc1, bc1r):
    return pl.pallas_call(
        _k_mid_body,
        grid=(T * N // _BM,),
        in_specs=[
            pl.BlockSpec((_BM, H), lambda i: (i, 0)),
            pl.BlockSpec((_BM, H), lambda i: (i, 0)),
            pl.BlockSpec((H, H), lambda i: (0, 0)),
            pl.BlockSpec((1, H), lambda i: (0, 0)),
        ],
        out_specs=pl.BlockSpec((_BM, H), lambda i: (i, 0)),
        out_shape=jax.ShapeDtypeStruct((T * N, H), jnp.float32),
    )(A1f, dinvf, Wc1, bc1r)


def _k_out_body(a_ref, dinv_ref, wc_ref, bc_ref, wih_ref, whh_ref,
                bih_ref, bhh_ref, wl_ref, bl_ref, o_ref):
    hprev = jnp.zeros((_BN, H), jnp.float32)
    for t in range(T):
        y = jnp.dot(a_ref[t], wc_ref[...],
                    preferred_element_type=jnp.float32)
        e = jnp.tanh(dinv_ref[t] * y + bc_ref[...])
        gi = jnp.dot(e, wih_ref[...],
                     preferred_element_type=jnp.float32) + bih_ref[...]
        gh = jnp.dot(hprev, whh_ref[...],
                     preferred_element_type=jnp.float32) + bhh_ref[...]
        r = jax.nn.sigmoid(gi[:, 0:H] + gh[:, 0:H])
        z = jax.nn.sigmoid(gi[:, H:2 * H] + gh[:, H:2 * H])
        n = jnp.tanh(gi[:, 2 * H:3 * H] + r * gh[:, 2 * H:3 * H])
        hprev = (1.0 - z) * n + z * hprev
        o_ref[t] = jnp.dot(hprev, wl_ref[...],
                           preferred_element_type=jnp.float32) + bl_ref[...]


def _k_out(A2, dinv32, Wc2, bc2r, WihT, WhhT, bihr, bhhr, WlT, blr):
    full = lambda shape: pl.BlockSpec(shape, lambda nb: tuple(0 for _ in shape))
    return pl.pallas_call(
        _k_out_body,
        grid=(N // _BN,),
        in_specs=[
            pl.BlockSpec((T, _BN, H), lambda nb: (0, nb, 0)),
            pl.BlockSpec((T, _BN, H), lambda nb: (0, nb, 0)),
            full((H, H)),
            full((1, H)),
            full((H, 3 * H)),
            full((H, 3 * H)),
            full((1, 3 * H)),
            full((1, 3 * H)),
            full((H, Z)),
            full((1, Z)),
        ],
        out_specs=pl.BlockSpec((T, _BN, Z), lambda nb: (0, nb, 0)),
        out_shape=jax.ShapeDtypeStruct((T, N, Z), jnp.float32),
    )(A2, dinv32, Wc2, bc2r, WihT, WhhT, bihr, bhhr, WlT, blr)


def kernel(xs, eis, W1, b1, Wc1, bc1, Wc2, bc2, W_ih, W_hh, b_ih, b_hh,
           Wl, bl):
    src_r = eis[:, 0, :].reshape(T, NS, NCH, C)
    dst_r = eis[:, 1, :].reshape(T, NS, NCH, C)
    ones = jnp.ones((RPT, DW), jnp.float32)

    deg16 = _sc_deg(dst_r, ones)
    h = _k_h(xs, W1, b1.reshape(1, H))
    dinv32, G1 = _k_g1(deg16, h)
    A1 = _sc_agg(src_r, dst_r, G1)
    G2 = _k_mid(A1.reshape(T * N, H), dinv32.reshape(T * N, H),
                Wc1, bc1.reshape(1, H)).reshape(T, N, H)
    A2 = _sc_agg(src_r, dst_r, G2)
    out = _k_out(A2, dinv32, Wc2, bc2.reshape(1, H),
                 W_ih.T, W_hh.T, b_ih.reshape(1, 3 * H),
                 b_hh.reshape(1, 3 * H), Wl.T, bl.reshape(1, Z))
    return out


# speedup vs baseline: 62.2147x; 2.3544x over previous
import functools

import jax
import jax.numpy as jnp
from jax import lax
from jax.experimental import pallas as pl
from jax.experimental.pallas import tpu as pltpu
from jax.experimental.pallas import tpu_sc as plsc

N = 10000
T = 12
E = 320000
X_DIM = 128
H = 32
Z = 16

NC = 2
NS = 16
TPC = T // NC
EPT = E // NS
C = 80
NCH = EPT // C
RPT = N // NS
DW = 16
NB = 5

_sc_mesh = plsc.VectorSubcoreMesh(core_axis_name="c", subcore_axis_name="s")
_sc_params = pltpu.CompilerParams(use_tc_tiling_on_sc=False)


@functools.partial(
    pl.kernel,
    out_type=jax.ShapeDtypeStruct((T, N, DW), jnp.float32),
    mesh=_sc_mesh,
    compiler_params=_sc_params,
    scratch_types=[
        pltpu.VMEM((NCH, C), jnp.int32),
        pltpu.VMEM((RPT, DW), jnp.float32),
        pltpu.VMEM_SHARED((N, DW), jnp.float32),
    ],
)
def _sc_deg(dst_hbm, ones_hbm, out_hbm, dst_v, ones_v, acc):
    c = lax.axis_index("c")
    s = lax.axis_index("s")
    rows = pl.ds(s * RPT, RPT)
    pltpu.sync_copy(ones_hbm, ones_v)

    def t_body(k, carry):
        t = c * TPC + k
        pltpu.sync_copy(dst_hbm.at[t, s], dst_v)
        pltpu.sync_copy(ones_v, acc.at[rows])
        plsc.subcore_barrier()

        def ch_body(j, carry2):
            pltpu.sync_copy(ones_v.at[pl.ds(0, C)], acc.at[dst_v.at[j]],
                            add=True)
            return carry2

        lax.fori_loop(0, NCH, ch_body, 0)
        plsc.subcore_barrier()
        pltpu.sync_copy(acc.at[rows], out_hbm.at[t, rows])
        return carry

    lax.fori_loop(0, TPC, t_body, 0)


@functools.partial(
    pl.kernel,
    out_type=jax.ShapeDtypeStruct((T, N, H), jnp.float32),
    mesh=_sc_mesh,
    compiler_params=_sc_params,
    scratch_types=[
        pltpu.VMEM((NCH, C), jnp.int32),
        pltpu.VMEM((NCH, C), jnp.int32),
        [pltpu.VMEM((C, H), jnp.float32) for _ in range(NB)],
        pltpu.VMEM_SHARED((N, H), jnp.float32),
        [pltpu.SemaphoreType.DMA for _ in range(NB)],
    ],
)
def _sc_agg(src_hbm, dst_hbm, g_hbm, out_hbm, src_v, dst_v, rows, acc, sems):
    c = lax.axis_index("c")
    s = lax.axis_index("s")
    own = pl.ds(s * RPT, RPT)

    def t_body(k, carry):
        t = c * TPC + k
        pltpu.sync_copy(src_hbm.at[t, s], src_v)
        pltpu.sync_copy(dst_hbm.at[t, s], dst_v)
        g_t = g_hbm.at[t]
        pltpu.sync_copy(g_t.at[own], acc.at[own])
        plsc.subcore_barrier()

        for b in range(NB - 1):
            pltpu.async_copy(g_t.at[src_v.at[b]], rows[b], sems[b])

        def blk_body(j2, carry2):
            for b in range(NB):
                j = j2 * NB + b
                nj = j + NB - 1
                nb_ = (b + NB - 1) % NB

                @pl.when(nj < NCH)
                def _():
                    pltpu.async_copy(g_t.at[src_v.at[nj]], rows[nb_],
                                     sems[nb_])

                pltpu.make_async_copy(g_t.at[src_v.at[j]], rows[b],
                                      sems[b]).wait()
                pltpu.sync_copy(rows[b], acc.at[dst_v.at[j]], add=True)
            return carry2

        lax.fori_loop(0, NCH // NB, blk_body, 0)
        plsc.subcore_barrier()
        pltpu.sync_copy(acc.at[own], out_hbm.at[t, own])
        return carry

    lax.fori_loop(0, TPC, t_body, 0)


def _k_h_body(xs_ref, w_ref, b_ref, o_ref):
    o_ref[...] = jnp.maximum(
        jnp.dot(xs_ref[...], w_ref[...], preferred_element_type=jnp.float32)
        + b_ref[...], 0.0)


def _k_h(xs, W1, b1r):
    return pl.pallas_call(
        _k_h_body,
        out_shape=jax.ShapeDtypeStruct((N, H), jnp.float32),
    )(xs, W1, b1r)


_BN = 1000


def _k_g1_body(deg_ref, h_ref, dinv_ref, g1_ref):
    d = deg_ref[0]
    d32 = jnp.concatenate([d, d], axis=-1)
    dinv = lax.rsqrt(d32)
    dinv_ref[0] = dinv
    g1_ref[0] = h_ref[...] * dinv


def _k_g1(deg16, h):
    return pl.pallas_call(
        _k_g1_body,
        grid=(T, N // _BN),
        in_specs=[
            pl.BlockSpec((1, _BN, DW), lambda t, nb: (t, nb, 0)),
            pl.BlockSpec((_BN, H), lambda t, nb: (nb, 0)),
        ],
        out_specs=[
            pl.BlockSpec((1, _BN, H), lambda t, nb: (t, nb, 0)),
            pl.BlockSpec((1, _BN, H), lambda t, nb: (t, nb, 0)),
        ],
        out_shape=[
            jax.ShapeDtypeStruct((T, N, H), jnp.float32),
            jax.ShapeDtypeStruct((T, N, H), jnp.float32),
        ],
    )(deg16, h)


_BM = 8000


def _k_mid_body(a_ref, dinv_ref, w_ref, b_ref, o_ref):
    dv = dinv_ref[...]
    y = jnp.dot(a_ref[...], w_ref[...], preferred_element_type=jnp.float32)
    o_ref[...] = jnp.maximum(dv * y + b_ref[...], 0.0) * dv


def _k_mid(A1f, dinvf, Wc1, bc1r):
    return pl.pallas_call(
        _k_mid_body,
        grid=(T * N // _BM,),
        in_specs=[
            pl.BlockSpec((_BM, H), lambda i: (i, 0)),
            pl.BlockSpec((_BM, H), lambda i: (i, 0)),
            pl.BlockSpec((H, H), lambda i: (0, 0)),
            pl.BlockSpec((1, H), lambda i: (0, 0)),
        ],
        out_specs=pl.BlockSpec((_BM, H), lambda i: (i, 0)),
        out_shape=jax.ShapeDtypeStruct((T * N, H), jnp.float32),
    )(A1f, dinvf, Wc1, bc1r)


def _k_out_body(a_ref, dinv_ref, wc_ref, bc_ref, wih_ref, whh_ref,
                bih_ref, bhh_ref, wl_ref, bl_ref, o_ref):
    hprev = jnp.zeros((_BN, H), jnp.float32)
    for t in range(T):
        y = jnp.dot(a_ref[t], wc_ref[...],
                    preferred_element_type=jnp.float32)
        e = jnp.tanh(dinv_ref[t] * y + bc_ref[...])
        gi = jnp.dot(e, wih_ref[...],
                     preferred_element_type=jnp.float32) + bih_ref[...]
        gh = jnp.dot(hprev, whh_ref[...],
                     preferred_element_type=jnp.float32) + bhh_ref[...]
        r = jax.nn.sigmoid(gi[:, 0:H] + gh[:, 0:H])
        z = jax.nn.sigmoid(gi[:, H:2 * H] + gh[:, H:2 * H])
        n = jnp.tanh(gi[:, 2 * H:3 * H] + r * gh[:, 2 * H:3 * H])
        hprev = (1.0 - z) * n + z * hprev
        o_ref[t] = jnp.dot(hprev, wl_ref[...],
                           preferred_element_type=jnp.float32) + bl_ref[...]


def _k_out(A2, dinv32, Wc2, bc2r, WihT, WhhT, bihr, bhhr, WlT, blr):
    full = lambda shape: pl.BlockSpec(shape, lambda nb: tuple(0 for _ in shape))
    return pl.pallas_call(
        _k_out_body,
        grid=(N // _BN,),
        in_specs=[
            pl.BlockSpec((T, _BN, H), lambda nb: (0, nb, 0)),
            pl.BlockSpec((T, _BN, H), lambda nb: (0, nb, 0)),
            full((H, H)),
            full((1, H)),
            full((H, 3 * H)),
            full((H, 3 * H)),
            full((1, 3 * H)),
            full((1, 3 * H)),
            full((H, Z)),
            full((1, Z)),
        ],
        out_specs=pl.BlockSpec((T, _BN, Z), lambda nb: (0, nb, 0)),
        out_shape=jax.ShapeDtypeStruct((T, N, Z), jnp.float32),
    )(A2, dinv32, Wc2, bc2r, WihT, WhhT, bihr, bhhr, WlT, blr)


def kernel(xs, eis, W1, b1, Wc1, bc1, Wc2, bc2, W_ih, W_hh, b_ih, b_hh,
           Wl, bl):
    src_r = eis[:, 0, :].reshape(T, NS, NCH, C)
    dst_r = eis[:, 1, :].reshape(T, NS, NCH, C)
    ones = jnp.ones((RPT, DW), jnp.float32)

    deg16 = _sc_deg(dst_r, ones)
    h = _k_h(xs, W1, b1.reshape(1, H))
    dinv32, G1 = _k_g1(deg16, h)
    A1 = _sc_agg(src_r, dst_r, G1)
    G2 = _k_mid(A1.reshape(T * N, H), dinv32.reshape(T * N, H),
                Wc1, bc1.reshape(1, H)).reshape(T, N, H)
    A2 = _sc_agg(src_r, dst_r, G2)
    out = _k_out(A2, dinv32, Wc2, bc2.reshape(1, H),
                 W_ih.T, W_hh.T, b_ih.reshape(1, 3 * H),
                 b_hh.reshape(1, 3 * H), Wl.T, bl.reshape(1, Z))
    return out


# speedup vs baseline: 71.9609x; 1.1567x over previous
import functools

import jax
import jax.numpy as jnp
from jax import lax
from jax.experimental import pallas as pl
from jax.experimental.pallas import tpu as pltpu
from jax.experimental.pallas import tpu_sc as plsc

N = 10000
T = 12
E = 320000
X_DIM = 128
H = 32
Z = 16

NC = 2
NS = 16
TPC = T // NC
EPT = E // NS
C = 80
NCH = EPT // C
RPT = N // NS
DW = 16
NB = 10
GD = 6
NBD = 5

_sc_mesh = plsc.VectorSubcoreMesh(core_axis_name="c", subcore_axis_name="s")
_sc_params = pltpu.CompilerParams(use_tc_tiling_on_sc=False)


@functools.partial(
    pl.kernel,
    out_type=jax.ShapeDtypeStruct((T, N, DW), jnp.float32),
    mesh=_sc_mesh,
    compiler_params=_sc_params,
    scratch_types=[
        pltpu.VMEM((NCH, C), jnp.int32),
        pltpu.VMEM((RPT, DW), jnp.float32),
        pltpu.VMEM_SHARED((N, DW), jnp.float32),
        [pltpu.SemaphoreType.DMA for _ in range(NBD)],
    ],
)
def _sc_deg(dst_hbm, ones_hbm, out_hbm, dst_v, ones_v, acc, sems):
    c = lax.axis_index("c")
    s = lax.axis_index("s")
    rows = pl.ds(s * RPT, RPT)
    ones_c = ones_v.at[pl.ds(0, C)]
    pltpu.sync_copy(ones_hbm, ones_v)

    def t_body(k, carry):
        t = c * TPC + k
        pltpu.sync_copy(dst_hbm.at[t, s], dst_v)
        pltpu.sync_copy(ones_v, acc.at[rows])
        plsc.subcore_barrier()

        def blk_body(j2, carry2):
            for b in range(NBD):
                j = j2 * NBD + b

                @pl.when(j >= NBD)
                def _():
                    pltpu.make_async_copy(ones_c, acc.at[dst_v.at[j]],
                                          sems[b]).wait()

                pltpu.async_copy(ones_c, acc.at[dst_v.at[j]], sems[b],
                                 add=True)
            return carry2

        lax.fori_loop(0, NCH // NBD, blk_body, 0)
        for b in range(NBD):
            pltpu.make_async_copy(ones_c, acc.at[dst_v.at[0]],
                                  sems[b]).wait()
        plsc.subcore_barrier()
        pltpu.sync_copy(acc.at[rows], out_hbm.at[t, rows])
        return carry

    lax.fori_loop(0, TPC, t_body, 0)


@functools.partial(
    pl.kernel,
    out_type=jax.ShapeDtypeStruct((T, N, H), jnp.float32),
    mesh=_sc_mesh,
    compiler_params=_sc_params,
    scratch_types=[
        pltpu.VMEM((NCH, C), jnp.int32),
        pltpu.VMEM((NCH, C), jnp.int32),
        [pltpu.VMEM((C, H), jnp.float32) for _ in range(NB)],
        pltpu.VMEM_SHARED((N, H), jnp.float32),
        [pltpu.SemaphoreType.DMA for _ in range(NB)],
        [pltpu.SemaphoreType.DMA for _ in range(NB)],
    ],
)
def _sc_agg(src_hbm, dst_hbm, g_hbm, out_hbm, src_v, dst_v, rows, acc,
            gsems, ssems):
    c = lax.axis_index("c")
    s = lax.axis_index("s")
    own = pl.ds(s * RPT, RPT)

    def t_body(k, carry):
        t = c * TPC + k
        pltpu.sync_copy(src_hbm.at[t, s], src_v)
        pltpu.sync_copy(dst_hbm.at[t, s], dst_v)
        g_t = g_hbm.at[t]
        pltpu.sync_copy(g_t.at[own], acc.at[own])
        plsc.subcore_barrier()

        for b in range(GD):
            pltpu.async_copy(g_t.at[src_v.at[b]], rows[b], gsems[b])

        def blk_body(j2, carry2):
            for b in range(NB):
                j = j2 * NB + b
                pj = j + GD
                pb = (b + GD) % NB

                @pl.when(pj < NCH)
                def _():
                    @pl.when(pj >= NB)
                    def _():
                        pltpu.make_async_copy(
                            rows[pb], acc.at[dst_v.at[0]], ssems[pb]).wait()

                    pltpu.async_copy(g_t.at[src_v.at[pj]], rows[pb],
                                     gsems[pb])

                pltpu.make_async_copy(g_t.at[src_v.at[j]], rows[b],
                                      gsems[b]).wait()
                pltpu.async_copy(rows[b], acc.at[dst_v.at[j]], ssems[b],
                                 add=True)
            return carry2

        lax.fori_loop(0, NCH // NB, blk_body, 0)
        for b in range(NB):
            pltpu.make_async_copy(rows[b], acc.at[dst_v.at[0]],
                                  ssems[b]).wait()
        plsc.subcore_barrier()
        pltpu.sync_copy(acc.at[own], out_hbm.at[t, own])
        return carry

    lax.fori_loop(0, TPC, t_body, 0)


def _k_h_body(xs_ref, w_ref, b_ref, o_ref):
    o_ref[...] = jnp.maximum(
        jnp.dot(xs_ref[...], w_ref[...], preferred_element_type=jnp.float32)
        + b_ref[...], 0.0)


def _k_h(xs, W1, b1r):
    return pl.pallas_call(
        _k_h_body,
        out_shape=jax.ShapeDtypeStruct((N, H), jnp.float32),
    )(xs, W1, b1r)


_BN = 1000


def _k_g1_body(deg_ref, h_ref, dinv_ref, g1_ref):
    d = deg_ref[0]
    d32 = jnp.concatenate([d, d], axis=-1)
    dinv = lax.rsqrt(d32)
    dinv_ref[0] = dinv
    g1_ref[0] = h_ref[...] * dinv


def _k_g1(deg16, h):
    return pl.pallas_call(
        _k_g1_body,
        grid=(T, N // _BN),
        in_specs=[
            pl.BlockSpec((1, _BN, DW), lambda t, nb: (t, nb, 0)),
            pl.BlockSpec((_BN, H), lambda t, nb: (nb, 0)),
        ],
        out_specs=[
            pl.BlockSpec((1, _BN, H), lambda t, nb: (t, nb, 0)),
            pl.BlockSpec((1, _BN, H), lambda t, nb: (t, nb, 0)),
        ],
        out_shape=[
            jax.ShapeDtypeStruct((T, N, H), jnp.float32),
            jax.ShapeDtypeStruct((T, N, H), jnp.float32),
        ],
    )(deg16, h)


_BM = 8000


def _k_mid_body(a_ref, dinv_ref, w_ref, b_ref, o_ref):
    dv = dinv_ref[...]
    y = jnp.dot(a_ref[...], w_ref[...], preferred_element_type=jnp.float32)
    o_ref[...] = jnp.maximum(dv * y + b_ref[...], 0.0) * dv


def _k_mid(A1f, dinvf, Wc1, bc1r):
    return pl.pallas_call(
        _k_mid_body,
        grid=(T * N // _BM,),
        in_specs=[
            pl.BlockSpec((_BM, H), lambda i: (i, 0)),
            pl.BlockSpec((_BM, H), lambda i: (i, 0)),
            pl.BlockSpec((H, H), lambda i: (0, 0)),
            pl.BlockSpec((1, H), lambda i: (0, 0)),
        ],
        out_specs=pl.BlockSpec((_BM, H), lambda i: (i, 0)),
        out_shape=jax.ShapeDtypeStruct((T * N, H), jnp.float32),
    )(A1f, dinvf, Wc1, bc1r)


def _k_out_body(a_ref, dinv_ref, wc_ref, bc_ref, wih_ref, whh_ref,
                bih_ref, bhh_ref, wl_ref, bl_ref, o_ref):
    hprev = jnp.zeros((_BN, H), jnp.float32)
    for t in range(T):
        y = jnp.dot(a_ref[t], wc_ref[...],
                    preferred_element_type=jnp.float32)
        e = jnp.tanh(dinv_ref[t] * y + bc_ref[...])
        gi = jnp.dot(e, wih_ref[...],
                     preferred_element_type=jnp.float32) + bih_ref[...]
        gh = jnp.dot(hprev, whh_ref[...],
                     preferred_element_type=jnp.float32) + bhh_ref[...]
        r = jax.nn.sigmoid(gi[:, 0:H] + gh[:, 0:H])
        z = jax.nn.sigmoid(gi[:, H:2 * H] + gh[:, H:2 * H])
        n = jnp.tanh(gi[:, 2 * H:3 * H] + r * gh[:, 2 * H:3 * H])
        hprev = (1.0 - z) * n + z * hprev
        o_ref[t] = jnp.dot(hprev, wl_ref[...],
                           preferred_element_type=jnp.float32) + bl_ref[...]


def _k_out(A2, dinv32, Wc2, bc2r, WihT, WhhT, bihr, bhhr, WlT, blr):
    full = lambda shape: pl.BlockSpec(shape, lambda nb: tuple(0 for _ in shape))
    return pl.pallas_call(
        _k_out_body,
        grid=(N // _BN,),
        in_specs=[
            pl.BlockSpec((T, _BN, H), lambda nb: (0, nb, 0)),
            pl.BlockSpec((T, _BN, H), lambda nb: (0, nb, 0)),
            full((H, H)),
            full((1, H)),
            full((H, 3 * H)),
            full((H, 3 * H)),
            full((1, 3 * H)),
            full((1, 3 * H)),
            full((H, Z)),
            full((1, Z)),
        ],
        out_specs=pl.BlockSpec((T, _BN, Z), lambda nb: (0, nb, 0)),
        out_shape=jax.ShapeDtypeStruct((T, N, Z), jnp.float32),
    )(A2, dinv32, Wc2, bc2r, WihT, WhhT, bihr, bhhr, WlT, blr)


def kernel(xs, eis, W1, b1, Wc1, bc1, Wc2, bc2, W_ih, W_hh, b_ih, b_hh,
           Wl, bl):
    src_r = eis[:, 0, :].reshape(T, NS, NCH, C)
    dst_r = eis[:, 1, :].reshape(T, NS, NCH, C)
    ones = jnp.ones((RPT, DW), jnp.float32)

    deg16 = _sc_deg(dst_r, ones)
    h = _k_h(xs, W1, b1.reshape(1, H))
    dinv32, G1 = _k_g1(deg16, h)
    A1 = _sc_agg(src_r, dst_r, G1)
    G2 = _k_mid(A1.reshape(T * N, H), dinv32.reshape(T * N, H),
                Wc1, bc1.reshape(1, H)).reshape(T, N, H)
    A2 = _sc_agg(src_r, dst_r, G2)
    out = _k_out(A2, dinv32, Wc2, bc2.reshape(1, H),
                 W_ih.T, W_hh.T, b_ih.reshape(1, 3 * H),
                 b_hh.reshape(1, 3 * H), Wl.T, bl.reshape(1, Z))
    return out
